# baseline (speedup 1.0000x reference)
import functools

import jax
import jax.numpy as jnp
from jax import lax
from jax.experimental import pallas as pl
from jax.experimental.pallas import tpu as pltpu
from jax.experimental.pallas import tpu_sc as plsc

B, L, D = 16384, 200, 16
LP = 208
NC, NS = 2, 16
NW = NC * NS
R = B // NW
CR = 8
NCH = R // CR


def _tree_sum(vs):
    while len(vs) > 1:
        vs = [vs[i] + vs[i + 1] for i in range(0, len(vs) - 1, 2)] + (
            [vs[-1]] if len(vs) % 2 else [])
    return vs[0]


def _body(x_hbm, m_hbm, out_hbm, xbuf, mbuf, obuf, sems):
    cid = lax.axis_index("c")
    sid = lax.axis_index("s")
    wid = sid * NC + cid
    base = wid * R

    def start(c, slot):
        pltpu.async_copy(x_hbm.at[pl.ds(base + c * CR, CR)], xbuf.at[slot],
                         sems.at[slot])
        pltpu.async_copy(m_hbm.at[pl.ds(base + c * CR, CR)], mbuf.at[slot],
                         sems.at[slot])

    def wait(c, slot):
        pltpu.make_async_copy(x_hbm.at[pl.ds(base + c * CR, CR)],
                              xbuf.at[slot], sems.at[slot]).wait()
        pltpu.make_async_copy(m_hbm.at[pl.ds(base + c * CR, CR)],
                              mbuf.at[slot], sems.at[slot]).wait()

    start(0, 0)
    start(1, 1)

    def process(c, slot):
        wait(c, slot)

        @pl.when(c + 2 < NCH)
        def _():
            start(c + 2, slot)

        for r in range(CR):
            def lstep(i, acc):
                l0 = i * 16
                mv = mbuf[slot, r, pl.ds(l0, 16)]
                prods = [xbuf[slot, r, l0 + u, :] * mv[u] for u in range(16)]
                return acc + _tree_sum(prods)

            acc = lax.fori_loop(0, L // 16, lstep, jnp.zeros((D,), jnp.float32))
            mv = mbuf[slot, r, pl.ds(12 * 16, 16)]
            prods = [xbuf[slot, r, 12 * 16 + u, :] * mv[u] for u in range(8)]
            obuf[r, :] = acc + _tree_sum(prods)
        pltpu.sync_copy(obuf, out_hbm.at[pl.ds(base + c * CR, CR)])

    def two_chunks(cp, _):
        process(2 * cp, 0)
        process(2 * cp + 1, 1)
        return 0

    lax.fori_loop(0, NCH // 2, two_chunks, 0)


@jax.jit
def _run(inputs, maskf):
    mesh = plsc.VectorSubcoreMesh(core_axis_name="c", subcore_axis_name="s")
    fn = functools.partial(
        pl.kernel,
        out_type=jax.ShapeDtypeStruct((B, D), jnp.float32),
        mesh=mesh,
        compiler_params=pltpu.CompilerParams(use_tc_tiling_on_sc=False),
        scratch_types=[
            pltpu.VMEM((2, CR, L, D), jnp.float32),
            pltpu.VMEM((2, CR, LP), jnp.float32),
            pltpu.VMEM((CR, D), jnp.float32),
            pltpu.SemaphoreType.DMA((2,)),
        ],
    )(_body)
    return fn(inputs, maskf)


def kernel(inputs, mask):
    maskf = jnp.pad(mask.astype(jnp.float32), ((0, 0), (0, LP - L)))
    return _run(inputs, maskf)
